# Initial kernel scaffold; baseline (speedup 1.0000x reference)
import jax
import jax.numpy as jnp
from jax.experimental import pallas as pl


def kernel(user_hashed_ids, item_hashed_ids, dense_features, sparse_features, user_tab, item_tab, tag_tab, wide_W, wide_b, W1, b1, W2, b2, W3, b3, W4, b4, tog_W, tog_b):
    raise NotImplementedError("write your pallas kernel here")


# speedup vs baseline: 1.3447x; 1.3447x over previous
import functools
import numpy as np
import jax
import jax.numpy as jnp
from jax import lax
from jax.experimental import pallas as pl
from jax.experimental.pallas import tpu as pltpu
from jax.experimental.pallas import tpu_sc as plsc

B = 4096
V = 100000
D = 64
DF = 16
NS = 26
NW = 32
BPW = B // NW
CH = 64
NCH = BPW // CH
SRC = CH * NS
NSTR = SRC // 128


def _sc_body(uids, iids, sf_flat, didx_hbm, zeros_hbm, utab, itab, ttab,
             ue, ie, te,
             uidx_v, iidx_v, tidx_v, didx_v, ubuf, ibuf, tbuf, acc,
             sem_u, sem_i, sem_t):
    cid = lax.axis_index("c")
    sid = lax.axis_index("s")
    wid = sid * 2 + cid
    base = wid * BPW
    soff = sid * BPW
    pltpu.sync_copy(uids.at[pl.ds(base, BPW)], uidx_v)
    pltpu.sync_copy(iids.at[pl.ds(base, BPW)], iidx_v)
    pltpu.sync_copy(didx_hbm.at[sid], didx_v)
    for c in range(NCH):
        cu = pltpu.async_copy(utab.at[uidx_v.at[pl.ds(c * CH, CH)]],
                              ubuf, sem_u)
        ci = pltpu.async_copy(itab.at[iidx_v.at[pl.ds(c * CH, CH)]],
                              ibuf, sem_i)
        pltpu.sync_copy(sf_flat.at[pl.ds((base + c * CH) * NS, SRC)], tidx_v)
        pltpu.sync_copy(zeros_hbm, acc.at[pl.ds(soff + c * CH, CH)])
        descs = []
        for s in range(NSTR):
            descs.append(pltpu.async_copy(
                ttab.at[tidx_v.at[pl.ds(s * 128, 128)]],
                tbuf.at[pl.ds(s * 128, 128)], sem_t))
        for dsc in descs:
            dsc.wait()
        for s in range(NSTR):
            pltpu.sync_copy(tbuf.at[pl.ds(s * 128, 128)],
                            acc.at[didx_v.at[c, s]], add=True)
        pltpu.sync_copy(acc.at[pl.ds(soff + c * CH, CH)],
                        te.at[pl.ds(base + c * CH, CH)])
        cu.wait()
        ci.wait()
        pltpu.sync_copy(ubuf, ue.at[pl.ds(base + c * CH, CH)])
        pltpu.sync_copy(ibuf, ie.at[pl.ds(base + c * CH, CH)])


_sc_gather = pl.kernel(
    _sc_body,
    out_type=(jax.ShapeDtypeStruct((B, D), jnp.float32),) * 3,
    mesh=plsc.VectorSubcoreMesh(core_axis_name="c", subcore_axis_name="s"),
    scratch_types=[
        pltpu.VMEM((BPW,), jnp.int32),
        pltpu.VMEM((BPW,), jnp.int32),
        pltpu.VMEM((SRC,), jnp.int32),
        pltpu.VMEM((NCH, NSTR, 128), jnp.int32),
        pltpu.VMEM((CH, D), jnp.float32),
        pltpu.VMEM((CH, D), jnp.float32),
        pltpu.VMEM((SRC, D), jnp.float32),
        pltpu.VMEM_SHARED((16 * BPW, D), jnp.float32),
        pltpu.SemaphoreType.DMA,
        pltpu.SemaphoreType.DMA,
        pltpu.SemaphoreType.DMA,
    ],
    compiler_params=pltpu.CompilerParams(use_tc_tiling_on_sc=False),
)

BM = 512


def _mlp_body(u, it, tg, dn, wW, wb, W1, b1, W2, b2, W3, b3, W4, b4, tW, tb,
              out):
    comb = jnp.concatenate([u[...], it[...], tg[...], dn[...]], axis=-1)
    dot = functools.partial(jnp.dot, preferred_element_type=jnp.float32,
                            precision=lax.Precision.HIGHEST)
    wide = dot(comb, wW[...]) + wb[...]
    h = jnp.maximum(dot(comb, W1[...]) + b1[...], 0.0)
    h = jnp.maximum(dot(h, W2[...]) + b2[...], 0.0)
    h = jnp.maximum(dot(h, W3[...]) + b3[...], 0.0)
    deep = dot(h, W4[...]) + b4[...]
    cat2 = jnp.concatenate([wide, deep], axis=-1)
    logit = dot(cat2, tW[...]) + tb[...]
    out[...] = jax.nn.sigmoid(logit)


def _full(shape):
    nd = len(shape)
    return pl.BlockSpec(shape, lambda i: (0,) * nd)


_mlp = pl.pallas_call(
    _mlp_body,
    grid=(B // BM,),
    in_specs=[
        pl.BlockSpec((BM, D), lambda i: (i, 0)),
        pl.BlockSpec((BM, D), lambda i: (i, 0)),
        pl.BlockSpec((BM, D), lambda i: (i, 0)),
        pl.BlockSpec((BM, DF), lambda i: (i, 0)),
        _full((3 * D + DF, D)), _full((D,)),
        _full((3 * D + DF, 2 * D)), _full((2 * D,)),
        _full((2 * D, 2 * D)), _full((2 * D,)),
        _full((2 * D, 2 * D)), _full((2 * D,)),
        _full((2 * D, D)), _full((D,)),
        _full((2 * D, 1)), _full((1,)),
    ],
    out_specs=pl.BlockSpec((BM, 1), lambda i: (i, 0)),
    out_shape=jax.ShapeDtypeStruct((B, 1), jnp.float32),
)

_DIDX = (np.arange(16)[:, None, None, None] * BPW
         + np.arange(NCH)[None, :, None, None] * CH
         + (np.arange(SRC) // NS).reshape(NSTR, 128)[None, None]
         ).astype(np.int32)


def kernel(user_hashed_ids, item_hashed_ids, dense_features, sparse_features,
           user_tab, item_tab, tag_tab,
           wide_W, wide_b, W1, b1, W2, b2, W3, b3, W4, b4, tog_W, tog_b):
    sf_flat = sparse_features.astype(jnp.int32).reshape(-1)
    uids = user_hashed_ids.astype(jnp.int32)
    iids = item_hashed_ids.astype(jnp.int32)
    didx = jnp.asarray(_DIDX)
    zeros = jnp.zeros((CH, D), jnp.float32)
    ue, ie, te = _sc_gather(uids, iids, sf_flat, didx, zeros,
                            user_tab, item_tab, tag_tab)
    return _mlp(ue, ie, te, dense_features,
                wide_W, wide_b, W1, b1, W2, b2, W3, b3, W4, b4, tog_W, tog_b)


# speedup vs baseline: 1.3761x; 1.0233x over previous
import functools
import numpy as np
import jax
import jax.numpy as jnp
from jax import lax
from jax.experimental import pallas as pl
from jax.experimental.pallas import tpu as pltpu
from jax.experimental.pallas import tpu_sc as plsc

B = 4096
V = 100000
D = 64
DF = 16
NS = 26
NW = 32
BPW = B // NW
CH = 32
NCH = BPW // CH
SRC = CH * NS
SSTR = 104
NSTR = SRC // SSTR


def _sc_body(uids, iids, sf_flat, up, ip, tp,
             ue, ie, te,
             uidx_v, iidx_v, tidx_v, ubuf, ibuf, tbuf, tebuf,
             sem_u, sem_i, sem_t):
    cid = lax.axis_index("c")
    sid = lax.axis_index("s")
    wid = sid * 2 + cid
    base = wid * BPW
    pltpu.sync_copy(uids.at[pl.ds(base, BPW)], uidx_v)
    pltpu.sync_copy(iids.at[pl.ds(base, BPW)], iidx_v)
    for c in range(NCH):
        pltpu.sync_copy(sf_flat.at[pl.ds((base + c * CH) * NS, SRC)], tidx_v)
        tds = [pltpu.async_copy(tp.at[tidx_v.at[pl.ds(s * SSTR, SSTR)]],
                                tbuf.at[pl.ds(s * SSTR, SSTR)], sem_t)
               for s in range(NSTR)]
        cu = pltpu.async_copy(up.at[uidx_v.at[pl.ds(c * CH, CH)]],
                              ubuf, sem_u)
        ci = pltpu.async_copy(ip.at[iidx_v.at[pl.ds(c * CH, CH)]],
                              ibuf, sem_i)
        for t in tds:
            t.wait()

        def pool_row(r, carry):
            for d in range(4):
                acc = tbuf[r * NS, pl.ds(d * 16, 16)]
                for j in range(1, NS):
                    acc = acc + tbuf[r * NS + j, pl.ds(d * 16, 16)]
                tebuf[r, pl.ds(d * 16, 16)] = acc
            return carry

        lax.fori_loop(0, CH, pool_row, 0)
        pltpu.sync_copy(tebuf, te.at[pl.ds(base + c * CH, CH)])
        cu.wait()
        ci.wait()
        pltpu.sync_copy(ubuf, ue.at[pl.ds(base + c * CH, CH)])
        pltpu.sync_copy(ibuf, ie.at[pl.ds(base + c * CH, CH)])


_sc_gather = pl.kernel(
    _sc_body,
    out_type=(jax.ShapeDtypeStruct((B, 2 * D), jnp.float32),) * 3,
    mesh=plsc.VectorSubcoreMesh(core_axis_name="c", subcore_axis_name="s"),
    scratch_types=[
        pltpu.VMEM((BPW,), jnp.int32),
        pltpu.VMEM((BPW,), jnp.int32),
        pltpu.VMEM((SRC,), jnp.int32),
        pltpu.VMEM((CH, 2 * D), jnp.float32),
        pltpu.VMEM((CH, 2 * D), jnp.float32),
        pltpu.VMEM((SRC, 2 * D), jnp.float32),
        pltpu.VMEM((CH, 2 * D), jnp.float32),
        pltpu.SemaphoreType.DMA,
        pltpu.SemaphoreType.DMA,
        pltpu.SemaphoreType.DMA,
    ],
    compiler_params=pltpu.CompilerParams(use_tc_tiling_on_sc=True),
)

BM = 512


def _mlp_body(u, it, tg, dn, wW, wb, W1, b1, W2, b2, W3, b3, W4, b4, tW, tb,
              out):
    comb = jnp.concatenate([u[...][:, :D], it[...][:, :D], tg[...][:, :D],
                            dn[...]], axis=-1)
    dot = functools.partial(jnp.dot, preferred_element_type=jnp.float32,
                            precision=lax.Precision.HIGHEST)
    wide = dot(comb, wW[...]) + wb[...]
    h = jnp.maximum(dot(comb, W1[...]) + b1[...], 0.0)
    h = jnp.maximum(dot(h, W2[...]) + b2[...], 0.0)
    h = jnp.maximum(dot(h, W3[...]) + b3[...], 0.0)
    deep = dot(h, W4[...]) + b4[...]
    cat2 = jnp.concatenate([wide, deep], axis=-1)
    logit = dot(cat2, tW[...]) + tb[...]
    out[...] = jax.nn.sigmoid(logit)


def _full(shape):
    nd = len(shape)
    return pl.BlockSpec(shape, lambda i: (0,) * nd)


_mlp = pl.pallas_call(
    _mlp_body,
    grid=(B // BM,),
    in_specs=[
        pl.BlockSpec((BM, 2 * D), lambda i: (i, 0)),
        pl.BlockSpec((BM, 2 * D), lambda i: (i, 0)),
        pl.BlockSpec((BM, 2 * D), lambda i: (i, 0)),
        pl.BlockSpec((BM, DF), lambda i: (i, 0)),
        _full((3 * D + DF, D)), _full((D,)),
        _full((3 * D + DF, 2 * D)), _full((2 * D,)),
        _full((2 * D, 2 * D)), _full((2 * D,)),
        _full((2 * D, 2 * D)), _full((2 * D,)),
        _full((2 * D, D)), _full((D,)),
        _full((2 * D, 1)), _full((1,)),
    ],
    out_specs=pl.BlockSpec((BM, 1), lambda i: (i, 0)),
    out_shape=jax.ShapeDtypeStruct((B, 1), jnp.float32),
)


def kernel(user_hashed_ids, item_hashed_ids, dense_features, sparse_features,
           user_tab, item_tab, tag_tab,
           wide_W, wide_b, W1, b1, W2, b2, W3, b3, W4, b4, tog_W, tog_b):
    sf_flat = sparse_features.astype(jnp.int32).reshape(-1)
    uids = user_hashed_ids.astype(jnp.int32)
    iids = item_hashed_ids.astype(jnp.int32)
    up = jnp.pad(user_tab, ((0, 0), (0, D)))
    ip = jnp.pad(item_tab, ((0, 0), (0, D)))
    tp = jnp.pad(tag_tab, ((0, 0), (0, D)))
    ue, ie, te = _sc_gather(uids, iids, sf_flat, up, ip, tp)
    return _mlp(ue, ie, te, dense_features,
                wide_W, wide_b, W1, b1, W2, b2, W3, b3, W4, b4, tog_W, tog_b)
